# Initial kernel scaffold; baseline (speedup 1.0000x reference)
import functools

import jax
import jax.numpy as jnp
from jax import lax
from jax.experimental import pallas as pl
from jax.experimental.pallas import tpu as pltpu
from jax.experimental.pallas import tpu_sc as plsc

EMBEDDING_DIM = 32
NUM_CORES = 2
NUM_SUBCORES = 16
NUM_WORKERS = NUM_CORES * NUM_SUBCORES
BATCH, SEQ = 16384, 50
B_TOTAL = BATCH * SEQ
B_PER_W = B_TOTAL // NUM_WORKERS
CHUNK = 128
N_CHUNKS = B_PER_W // CHUNK

_mesh = plsc.VectorSubcoreMesh(core_axis_name="c", subcore_axis_name="s")


@functools.partial(
    pl.kernel,
    mesh=_mesh,
    out_type=jax.ShapeDtypeStruct((B_TOTAL, EMBEDDING_DIM), jnp.float32),
    scratch_types=[
        pltpu.VMEM((N_CHUNKS, CHUNK), jnp.int32),
        pltpu.VMEM((CHUNK, EMBEDDING_DIM), jnp.float32),
        pltpu.SemaphoreType.DMA,
    ],
)
def _gather_kernel(idx_hbm, table_hbm, out_hbm, idx_v, rows_v, sem):
    wid = lax.axis_index("s") * NUM_CORES + lax.axis_index("c")
    base = wid * B_PER_W
    pltpu.sync_copy(idx_hbm.at[wid], idx_v)

    def body(j, carry):
        pltpu.async_copy(table_hbm.at[idx_v.at[j]], rows_v, sem).wait()
        pltpu.sync_copy(rows_v, out_hbm.at[pl.ds(base + j * CHUNK, CHUNK)])
        return carry

    lax.fori_loop(0, N_CHUNKS, body, 0)


def kernel(token_ids, weight):
    idx = token_ids.reshape(NUM_WORKERS, N_CHUNKS, CHUNK).astype(jnp.int32)
    out = _gather_kernel(idx, weight)
    return out.reshape(BATCH, SEQ, EMBEDDING_DIM)


# speedup vs baseline: 1.0227x; 1.0227x over previous
import functools

import jax
import jax.numpy as jnp
from jax import lax
from jax.experimental import pallas as pl
from jax.experimental.pallas import tpu as pltpu
from jax.experimental.pallas import tpu_sc as plsc

EMBEDDING_DIM = 32
NUM_CORES = 2
NUM_SUBCORES = 16
NUM_WORKERS = NUM_CORES * NUM_SUBCORES
BATCH, SEQ = 16384, 50
B_TOTAL = BATCH * SEQ
B_PER_W = B_TOTAL // NUM_WORKERS
CHUNK = 128
N_CHUNKS = B_PER_W // CHUNK

_mesh = plsc.VectorSubcoreMesh(core_axis_name="c", subcore_axis_name="s")


@functools.partial(
    pl.kernel,
    mesh=_mesh,
    out_type=jax.ShapeDtypeStruct((B_TOTAL, EMBEDDING_DIM), jnp.float32),
    scratch_types=[
        pltpu.VMEM((N_CHUNKS, CHUNK), jnp.int32),
        pltpu.VMEM((CHUNK, EMBEDDING_DIM), jnp.float32),
        pltpu.SemaphoreType.DMA,
    ],
    compiler_params=pltpu.CompilerParams(use_tc_tiling_on_sc=False),
)
def _gather_kernel(idx_hbm, table_hbm, out_hbm, idx_v, rows_v, sem):
    wid = lax.axis_index("s") * NUM_CORES + lax.axis_index("c")
    base = wid * B_PER_W
    pltpu.sync_copy(idx_hbm.at[wid], idx_v)

    def body(j, carry):
        pltpu.async_copy(table_hbm.at[idx_v.at[j]], rows_v, sem).wait()
        pltpu.sync_copy(rows_v, out_hbm.at[pl.ds(base + j * CHUNK, CHUNK)])
        return carry

    lax.fori_loop(0, N_CHUNKS, body, 0)


def kernel(token_ids, weight):
    idx = token_ids.reshape(NUM_WORKERS, N_CHUNKS, CHUNK).astype(jnp.int32)
    out = _gather_kernel(idx, weight)
    return out.reshape(BATCH, SEQ, EMBEDDING_DIM)


# speedup vs baseline: 1.1138x; 1.0891x over previous
import functools

import jax
import jax.numpy as jnp
from jax import lax
from jax.experimental import pallas as pl
from jax.experimental.pallas import tpu as pltpu
from jax.experimental.pallas import tpu_sc as plsc

EMBEDDING_DIM = 32
NUM_CORES = 2
NUM_SUBCORES = 16
NUM_WORKERS = NUM_CORES * NUM_SUBCORES
BATCH, SEQ = 16384, 50
B_TOTAL = BATCH * SEQ
B_PER_W = B_TOTAL // NUM_WORKERS
CHUNK = 128
N_CHUNKS = B_PER_W // CHUNK
NBUF = 8
DEPTH = 6

_mesh = plsc.VectorSubcoreMesh(core_axis_name="c", subcore_axis_name="s")


@functools.partial(
    pl.kernel,
    mesh=_mesh,
    out_type=jax.ShapeDtypeStruct((B_TOTAL, EMBEDDING_DIM), jnp.float32),
    scratch_types=[
        pltpu.VMEM((N_CHUNKS, CHUNK), jnp.int32),
        pltpu.VMEM((NBUF, CHUNK, EMBEDDING_DIM), jnp.float32),
        pltpu.SemaphoreType.DMA,
        pltpu.SemaphoreType.DMA,
    ],
    compiler_params=pltpu.CompilerParams(use_tc_tiling_on_sc=False),
)
def _gather_kernel(idx_hbm, table_hbm, out_hbm, idx_v, bufs, gsem, wsem):
    wid = lax.axis_index("s") * NUM_CORES + lax.axis_index("c")
    base = wid * B_PER_W
    pltpu.sync_copy(idx_hbm.at[wid], idx_v)

    def gather(j, b):
        return pltpu.make_async_copy(table_hbm.at[idx_v.at[j]], bufs.at[b], gsem)

    def write(j, b):
        return pltpu.make_async_copy(
            bufs.at[b], out_hbm.at[pl.ds(base + j * CHUNK, CHUNK)], wsem
        )

    for j in range(DEPTH):
        gather(j, j).start()

    def body(j, carry):
        b = lax.rem(j, NBUF)
        gather(j, b).wait()
        write(j, b).start()

        @pl.when(j + DEPTH < N_CHUNKS)
        def _():
            jn = j + DEPTH
            bn = lax.rem(jn, NBUF)

            @pl.when(jn >= NBUF)
            def _():
                jw = jn - NBUF
                write(jw, bn).wait()

            gather(jn, bn).start()

        return carry

    lax.fori_loop(0, N_CHUNKS, body, 0)

    for jw in range(N_CHUNKS - NBUF, N_CHUNKS):
        write(jw, jw % NBUF).wait()


def kernel(token_ids, weight):
    idx = token_ids.reshape(NUM_WORKERS, N_CHUNKS, CHUNK).astype(jnp.int32)
    out = _gather_kernel(idx, weight)
    return out.reshape(BATCH, SEQ, EMBEDDING_DIM)


# speedup vs baseline: 1.1150x; 1.0010x over previous
import functools

import jax
import jax.numpy as jnp
from jax import lax
from jax.experimental import pallas as pl
from jax.experimental.pallas import tpu as pltpu
from jax.experimental.pallas import tpu_sc as plsc

EMBEDDING_DIM = 32
NUM_CORES = 2
NUM_SUBCORES = 16
NUM_WORKERS = NUM_CORES * NUM_SUBCORES
BATCH, SEQ = 16384, 50
B_TOTAL = BATCH * SEQ
B_PER_W = B_TOTAL // NUM_WORKERS
CHUNK = 512
N_CHUNKS = B_PER_W // CHUNK
NBUF = 4
DEPTH = 3

_mesh = plsc.VectorSubcoreMesh(core_axis_name="c", subcore_axis_name="s")


@functools.partial(
    pl.kernel,
    mesh=_mesh,
    out_type=jax.ShapeDtypeStruct((B_TOTAL, EMBEDDING_DIM), jnp.float32),
    scratch_types=[
        pltpu.VMEM((N_CHUNKS, CHUNK), jnp.int32),
        pltpu.VMEM((NBUF, CHUNK, EMBEDDING_DIM), jnp.float32),
        pltpu.SemaphoreType.DMA,
        pltpu.SemaphoreType.DMA,
    ],
    compiler_params=pltpu.CompilerParams(use_tc_tiling_on_sc=False),
)
def _gather_kernel(idx_hbm, table_hbm, out_hbm, idx_v, bufs, gsem, wsem):
    wid = lax.axis_index("s") * NUM_CORES + lax.axis_index("c")
    base = wid * B_PER_W
    pltpu.sync_copy(idx_hbm.at[wid], idx_v)

    def gather(j, b):
        return pltpu.make_async_copy(table_hbm.at[idx_v.at[j]], bufs.at[b], gsem)

    def write(j, b):
        return pltpu.make_async_copy(
            bufs.at[b], out_hbm.at[pl.ds(base + j * CHUNK, CHUNK)], wsem
        )

    for j in range(DEPTH):
        gather(j, j).start()

    def body(j, carry):
        b = lax.rem(j, NBUF)
        gather(j, b).wait()
        write(j, b).start()

        @pl.when(j + DEPTH < N_CHUNKS)
        def _():
            jn = j + DEPTH
            bn = lax.rem(jn, NBUF)

            @pl.when(jn >= NBUF)
            def _():
                jw = jn - NBUF
                write(jw, bn).wait()

            gather(jn, bn).start()

        return carry

    lax.fori_loop(0, N_CHUNKS, body, 0)

    for jw in range(N_CHUNKS - NBUF, N_CHUNKS):
        write(jw, jw % NBUF).wait()


def kernel(token_ids, weight):
    idx = token_ids.reshape(NUM_WORKERS, N_CHUNKS, CHUNK).astype(jnp.int32)
    out = _gather_kernel(idx, weight)
    return out.reshape(BATCH, SEQ, EMBEDDING_DIM)


# speedup vs baseline: 1.8076x; 1.6212x over previous
import functools

import jax
import jax.numpy as jnp
from jax import lax
from jax.experimental import pallas as pl
from jax.experimental.pallas import tpu as pltpu
from jax.experimental.pallas import tpu_sc as plsc

EMBEDDING_DIM = 32
NUM_CORES = 2
NUM_SUBCORES = 16
NUM_WORKERS = NUM_CORES * NUM_SUBCORES
BATCH, SEQ = 16384, 50
ROWS_PER_W = BATCH // NUM_WORKERS
RPC = 8
N_CHUNKS = ROWS_PER_W // RPC
NBUF = 6
DEPTH = 4

_mesh = plsc.VectorSubcoreMesh(core_axis_name="c", subcore_axis_name="s")


@functools.partial(
    pl.kernel,
    mesh=_mesh,
    out_type=jax.ShapeDtypeStruct((BATCH, SEQ, EMBEDDING_DIM), jnp.float32),
    scratch_types=[
        pltpu.VMEM((N_CHUNKS, RPC, SEQ), jnp.int32),
        pltpu.VMEM((NBUF, RPC, SEQ, EMBEDDING_DIM), jnp.float32),
        pltpu.SemaphoreType.DMA,
        pltpu.SemaphoreType.DMA,
    ],
    compiler_params=pltpu.CompilerParams(use_tc_tiling_on_sc=False),
)
def _gather_kernel(idx_hbm, table_hbm, out_hbm, idx_v, bufs, gsem, wsem):
    wid = lax.axis_index("s") * NUM_CORES + lax.axis_index("c")
    row_base = wid * ROWS_PER_W
    pltpu.sync_copy(idx_hbm.at[wid], idx_v)

    def gather(j, r, b):
        return pltpu.make_async_copy(
            table_hbm.at[idx_v.at[j, r]], bufs.at[b, r], gsem
        )

    def start_gathers(j, b):
        for r in range(RPC):
            gather(j, r, b).start()

    def wait_gathers(j, b):
        for r in range(RPC):
            gather(j, r, b).wait()

    def write(j, b):
        dst = out_hbm.at[pl.ds(row_base + j * RPC, RPC)]
        return pltpu.make_async_copy(bufs.at[b], dst, wsem)

    for j in range(DEPTH):
        start_gathers(j, j)

    def body(j, carry):
        b = lax.rem(j, NBUF)
        wait_gathers(j, b)
        write(j, b).start()

        @pl.when(j + DEPTH < N_CHUNKS)
        def _():
            jn = j + DEPTH
            bn = lax.rem(jn, NBUF)

            @pl.when(jn >= NBUF)
            def _():
                write(jn - NBUF, bn).wait()

            start_gathers(jn, bn)

        return carry

    lax.fori_loop(0, N_CHUNKS, body, 0)

    for jw in range(N_CHUNKS - NBUF, N_CHUNKS):
        write(jw, jw % NBUF).wait()


def kernel(token_ids, weight):
    idx = token_ids.reshape(NUM_WORKERS, N_CHUNKS, RPC, SEQ).astype(jnp.int32)
    return _gather_kernel(idx, weight)


# speedup vs baseline: 1.8138x; 1.0034x over previous
import functools

import jax
import jax.numpy as jnp
from jax import lax
from jax.experimental import pallas as pl
from jax.experimental.pallas import tpu as pltpu
from jax.experimental.pallas import tpu_sc as plsc

EMBEDDING_DIM = 32
NUM_CORES = 2
NUM_SUBCORES = 16
NUM_WORKERS = NUM_CORES * NUM_SUBCORES
BATCH, SEQ = 16384, 50
ROWS_PER_W = BATCH // NUM_WORKERS
RPC = 8
N_CHUNKS = ROWS_PER_W // RPC
NBUF = 6
DEPTH = 4

_mesh = plsc.VectorSubcoreMesh(core_axis_name="c", subcore_axis_name="s")


@functools.partial(
    pl.kernel,
    mesh=_mesh,
    out_type=jax.ShapeDtypeStruct((BATCH, SEQ, EMBEDDING_DIM), jnp.float32),
    scratch_types=[
        pltpu.VMEM((ROWS_PER_W, SEQ), jnp.int32),
        pltpu.VMEM((NBUF, RPC, SEQ, EMBEDDING_DIM), jnp.float32),
        pltpu.SemaphoreType.DMA,
        pltpu.SemaphoreType.DMA,
    ],
    compiler_params=pltpu.CompilerParams(use_tc_tiling_on_sc=False),
)
def _gather_kernel(idx_hbm, table_hbm, out_hbm, idx_v, bufs, gsem, wsem):
    wid = lax.axis_index("s") * NUM_CORES + lax.axis_index("c")
    row_base = wid * ROWS_PER_W
    pltpu.sync_copy(idx_hbm.at[pl.ds(row_base, ROWS_PER_W)], idx_v)

    def gather(j, r, b):
        return pltpu.make_async_copy(
            table_hbm.at[idx_v.at[j * RPC + r]], bufs.at[b, r], gsem
        )

    def start_gathers(j, b):
        for r in range(RPC):
            gather(j, r, b).start()

    def wait_gathers(j, b):
        for r in range(RPC):
            gather(j, r, b).wait()

    def write(j, b):
        dst = out_hbm.at[pl.ds(row_base + j * RPC, RPC)]
        return pltpu.make_async_copy(bufs.at[b], dst, wsem)

    for j in range(DEPTH):
        start_gathers(j, j)

    def body(j, carry):
        b = lax.rem(j, NBUF)
        wait_gathers(j, b)
        write(j, b).start()

        @pl.when(j + DEPTH < N_CHUNKS)
        def _():
            jn = j + DEPTH
            bn = lax.rem(jn, NBUF)

            @pl.when(jn >= NBUF)
            def _():
                write(jn - NBUF, bn).wait()

            start_gathers(jn, bn)

        return carry

    lax.fori_loop(0, N_CHUNKS, body, 0)

    for jw in range(N_CHUNKS - NBUF, N_CHUNKS):
        write(jw, jw % NBUF).wait()


def kernel(token_ids, weight):
    return _gather_kernel(token_ids, weight)
